# baseline (speedup 1.0000x reference)
import functools
import jax
import jax.numpy as jnp
from jax import lax
from jax.experimental import pallas as pl
from jax.experimental.pallas import tpu as pltpu
from jax.experimental.pallas import tpu_sc as plsc

N = 10000
E = 320000
D = 128
B = 64
NCLS = 10
LAYERS = 2
EPS = 1e-5

NCORES = 2
NSUB = 16
NW = NCORES * NSUB
EW = E // NW
CH = 400
NCH = EW // CH

CH1 = 200
NCH1 = EW // CH1

SE_CHUNK = 8000
SE_NPASS = E // (SE_CHUNK * NCORES)
GARB = 192
TE = E // NSUB
TCH = TE // CH

EBLK = 2000
NEBLK = E // EBLK

@functools.lru_cache(maxsize=None)
def _mesh():
    return plsc.VectorSubcoreMesh(core_axis_name="c", subcore_axis_name="s")


def _zero2d(buf, rows):
    z16 = jnp.zeros((16,), jnp.float32)

    def body(r, _):
        for t in range(D // 16):
            buf[r, pl.ds(t * 16, 16)] = z16
        return 0

    lax.fori_loop(0, rows, body, 0)


def _zero1d(buf, n):
    z16 = jnp.zeros((16,), jnp.float32)

    def body(i, _):
        buf[pl.ds(i * 16, 16)] = z16
        return 0

    lax.fori_loop(0, n // 16, body, 0)


def _fill1d(buf, n, val):
    v16 = jnp.full((16,), val, jnp.float32)

    def body(i, _):
        buf[pl.ds(i * 16, 16)] = v16
        return 0

    lax.fori_loop(0, n // 16, body, 0)


def _relu_stats_inplace(buf, acc, rows):
    def body(r, _):
        for t in range(D // 16):
            y = jnp.maximum(buf[r, pl.ds(t * 16, 16)], 0.0)
            buf[r, pl.ds(t * 16, 16)] = y
            acc[pl.ds(t * 16, 16)] += y
            acc[pl.ds(D + t * 16, 16)] += y * y
        return 0

    lax.fori_loop(0, rows, body, 0)


@functools.lru_cache(maxsize=None)
def _sc_prep_k():
    return pl.kernel(
        _sc_prep,
        out_type=[
            jax.ShapeDtypeStruct((E,), jnp.int32),
            jax.ShapeDtypeStruct((NCORES * N,), jnp.float32),
            jax.ShapeDtypeStruct((NCORES * E,), jnp.float32),
        ],
        mesh=_mesh(),
        scratch_types=[
            pltpu.VMEM((CH,), jnp.int32),
            pltpu.VMEM((CH,), jnp.int32),
            pltpu.VMEM((CH,), jnp.int32),
            pltpu.VMEM((CH,), jnp.float32),
            pltpu.VMEM((20000,), jnp.float32),
            pltpu.VMEM((4000,), jnp.float32),
            pltpu.VMEM_SHARED((E,), jnp.float32),
            pltpu.VMEM_SHARED((N,), jnp.float32),
            pltpu.SemaphoreType.DMA,
        ],
    )


def _sc_prep(u_hbm, ed_hbm, v_hbm, sv_out, cntv_out, cnte_out,
             edbuf, vbuf, svbuf, ones, bounce, z1d, cnte_sh, cntv_sh, sem):
    cid = lax.axis_index("c")
    sid = lax.axis_index("s")
    wid = sid * NCORES + cid

    _fill1d(ones, CH, 1.0)
    _zero1d(z1d, 4000)
    for j in range(5):
        pltpu.sync_copy(z1d, cnte_sh.at[pl.ds(sid * 20000 + j * 4000, 4000)])

    @pl.when(sid == 0)
    def _():
        pltpu.sync_copy(z1d, cntv_sh.at[pl.ds(0, 4000)])
        pltpu.sync_copy(z1d, cntv_sh.at[pl.ds(4000, 4000)])
        pltpu.sync_copy(z1d.at[pl.ds(0, 2000)], cntv_sh.at[pl.ds(8000, 2000)])

    plsc.subcore_barrier()

    def chunk(k, _):
        base = wid * EW + k * CH
        pltpu.sync_copy(ed_hbm.at[pl.ds(base, CH)], edbuf)
        pltpu.async_copy(u_hbm.at[edbuf], svbuf, sem).wait()
        pltpu.sync_copy(svbuf, sv_out.at[pl.ds(base, CH)])
        pltpu.sync_copy(ones, cnte_sh.at[edbuf], add=True)
        pltpu.sync_copy(v_hbm.at[pl.ds(base, CH)], vbuf)
        pltpu.sync_copy(ones, cntv_sh.at[vbuf], add=True)
        return 0

    lax.fori_loop(0, NCH, chunk, 0)
    plsc.subcore_barrier()

    pltpu.sync_copy(cnte_sh.at[pl.ds(sid * 20000, 20000)], bounce)
    pltpu.sync_copy(bounce, cnte_out.at[pl.ds(cid * E + sid * 20000, 20000)])

    @pl.when(sid == 0)
    def _():
        pltpu.sync_copy(cntv_sh, bounce.at[pl.ds(0, N)])
        pltpu.sync_copy(bounce.at[pl.ds(0, N)], cntv_out.at[pl.ds(cid * N, N)])


@functools.lru_cache(maxsize=None)
def _sc_y1_k():
    return pl.kernel(
        _sc_y1,
        out_type=[
            jax.ShapeDtypeStruct((NCORES, N, D), jnp.float32),
            jax.ShapeDtypeStruct((NW, 1, 2 * D), jnp.float32),
        ],
        mesh=_mesh(),
        scratch_types=[
            pltpu.VMEM((CH1, D), jnp.float32),
            pltpu.VMEM((CH1,), jnp.int32),
            pltpu.VMEM((CH1,), jnp.int32),
            pltpu.VMEM((2 * D,), jnp.float32),
            pltpu.VMEM_SHARED((N, D), jnp.float32),
            pltpu.SemaphoreType.DMA,
        ],
    )


def _sc_y1(z1_hbm, t1_hbm, u_hbm, v_hbm, sv_out, st_out,
           zbuf, ubuf, vbuf, acc, sv_sh, sem):
    cid = lax.axis_index("c")
    sid = lax.axis_index("s")
    wid = sid * NCORES + cid

    _zero2d(zbuf, CH1)
    _zero1d(acc, 2 * D)

    @pl.when(sid < 10)
    def _():
        for j in range(1000 // CH1):
            pltpu.sync_copy(zbuf, sv_sh.at[pl.ds(sid * 1000 + j * CH1, CH1)])

    plsc.subcore_barrier()

    def chunk(k, _):
        base = wid * EW + k * CH1
        pltpu.sync_copy(z1_hbm.at[pl.ds(base, CH1)], zbuf)
        pltpu.sync_copy(u_hbm.at[pl.ds(base, CH1)], ubuf)
        pltpu.async_copy(t1_hbm.at[ubuf], zbuf, sem, add=True).wait()
        _relu_stats_inplace(zbuf, acc, CH1)
        pltpu.sync_copy(v_hbm.at[pl.ds(base, CH1)], vbuf)
        pltpu.sync_copy(zbuf, sv_sh.at[vbuf], add=True)
        return 0

    lax.fori_loop(0, NCH1, chunk, 0)
    pltpu.sync_copy(acc, st_out.at[wid, 0])
    plsc.subcore_barrier()

    @pl.when(sid < 10)
    def _():
        for j in range(1000 // CH1):
            pltpu.sync_copy(sv_sh.at[pl.ds(sid * 1000 + j * CH1, CH1)], zbuf)
            pltpu.sync_copy(zbuf, sv_out.at[cid, pl.ds(sid * 1000 + j * CH1, CH1)])


@functools.lru_cache(maxsize=None)
def _sc_y2_k():
    return pl.kernel(
        _sc_y2,
        out_type=[
            jax.ShapeDtypeStruct((E, D), jnp.float32),
            jax.ShapeDtypeStruct((NW, 1, 2 * D), jnp.float32),
        ],
        mesh=_mesh(),
        scratch_types=[
            pltpu.VMEM((CH, D), jnp.float32),
            pltpu.VMEM((CH,), jnp.int32),
            pltpu.VMEM((CH,), jnp.int32),
            pltpu.VMEM((2 * D,), jnp.float32),
            pltpu.SemaphoreType.DMA,
        ],
    )


def _sc_y2(z2_hbm, t2_hbm, es_hbm, sv_hbm, y2_out, st_out,
           zbuf, ibuf, ibuf2, acc, sem):
    cid = lax.axis_index("c")
    sid = lax.axis_index("s")
    wid = sid * NCORES + cid
    _zero1d(acc, 2 * D)

    def chunk(k, _):
        base = wid * EW + k * CH
        pltpu.sync_copy(es_hbm.at[pl.ds(base, CH)], ibuf)
        pltpu.sync_copy(sv_hbm.at[pl.ds(base, CH)], ibuf2)
        pltpu.async_copy(z2_hbm.at[ibuf], zbuf, sem).wait()
        pltpu.async_copy(t2_hbm.at[ibuf2], zbuf, sem, add=True).wait()
        _relu_stats_inplace(zbuf, acc, CH)
        pltpu.sync_copy(zbuf, y2_out.at[pl.ds(base, CH)])
        return 0

    lax.fori_loop(0, NCH, chunk, 0)
    pltpu.sync_copy(acc, st_out.at[wid, 0])


@functools.lru_cache(maxsize=None)
def _sc_scatter_e_k():
    return pl.kernel(
        _sc_scatter_e,
        out_type=jax.ShapeDtypeStruct((E, D), jnp.float32),
        mesh=_mesh(),
        scratch_types=[
            pltpu.VMEM((CH, D), jnp.float32),
            pltpu.VMEM((CH,), jnp.int32),
            pltpu.VMEM((CH,), jnp.int32),
            pltpu.VMEM_SHARED((SE_CHUNK + GARB, D), jnp.float32),
            pltpu.SemaphoreType.DMA,
        ],
    )


def _sc_scatter_e(y2_hbm, ed_hbm, se_out,
                  ybuf, edbuf, idxbuf, dest_sh, sem):
    cid = lax.axis_index("c")
    sid = lax.axis_index("s")

    def one_pass(p, _):
        _zero2d(ybuf, CH)
        zb = sid * 512
        pltpu.sync_copy(ybuf, dest_sh.at[pl.ds(zb, CH)])
        pltpu.sync_copy(ybuf.at[pl.ds(0, 112)],
                        dest_sh.at[pl.ds(zb + CH, 112)])
        plsc.subcore_barrier()

        lo = (p * NCORES + cid) * SE_CHUNK

        def chunk(k, _):
            ebase = sid * TE + k * CH
            pltpu.sync_copy(y2_hbm.at[pl.ds(ebase, CH)], ybuf)
            pltpu.sync_copy(ed_hbm.at[pl.ds(ebase, CH)], edbuf)

            def lanes(t, _):
                e = edbuf[pl.ds(t * 16, 16)]
                inb = jnp.logical_and(e >= lo, e < lo + SE_CHUNK)
                garb = SE_CHUNK + lax.rem(e, GARB)
                idxbuf[pl.ds(t * 16, 16)] = jnp.where(inb, e - lo, garb)
                return 0

            lax.fori_loop(0, CH // 16, lanes, 0)
            pltpu.sync_copy(ybuf, dest_sh.at[idxbuf], add=True)
            return 0

        lax.fori_loop(0, TCH, chunk, 0)
        plsc.subcore_barrier()

        @pl.when(sid < 10)
        def _():
            for off in (0, CH):
                pltpu.sync_copy(dest_sh.at[pl.ds(sid * 800 + off, CH)], ybuf)
                pltpu.sync_copy(ybuf, se_out.at[pl.ds(lo + sid * 800 + off, CH)])

        plsc.subcore_barrier()
        return 0

    lax.fori_loop(0, SE_NPASS, one_pass, 0)


def _r2c(row):
    ii = lax.broadcasted_iota(jnp.int32, (D, D), 0)
    jj = lax.broadcasted_iota(jnp.int32, (D, D), 1)
    return jnp.sum(jnp.where(ii == jj, row, 0.0), axis=1, keepdims=True)


def _c2r(col):
    ii = lax.broadcasted_iota(jnp.int32, (D, D), 0)
    jj = lax.broadcasted_iota(jnp.int32, (D, D), 1)
    return jnp.sum(jnp.where(ii == jj, col, 0.0), axis=0, keepdims=True)


def _bn_affine(stats, g_row, be_row, nrows):
    s = jnp.sum(stats.reshape(-1, 2 * D), axis=0, keepdims=True)
    me = s[:, :D] / nrows
    ve = s[:, D:] / nrows - me * me
    a = g_row / jnp.sqrt(ve + EPS)
    return a, be_row - a * me


def _full(shape):
    nd = len(shape)
    return pl.BlockSpec(shape, lambda: (0,) * nd)


def _tc_prep_body(first):
    def body(zv_ref, avgv_ref, ste_ref, pg_ref, pbe_ref,
             vuw_ref, vub_ref, edw_ref, edb_ref,
             t1_ref, t2_ref, wz1_ref, wz2_ref, c1_ref, c2_ref, aege_ref):
        av_c = avgv_ref[:, 0:1]
        gv_c = avgv_ref[:, 1:2]
        if first:
            ae = jnp.ones((1, D), jnp.float32)
            ge = jnp.zeros((1, D), jnp.float32)
        else:
            ae, ge = _bn_affine(ste_ref[...], pg_ref[...], pbe_ref[...], E)
        ae_c = _r2c(ae)
        ge_c = _r2c(ge)
        zv = zv_ref[...]
        wa = vuw_ref[:D, :]
        wb = vuw_ref[D:, :]
        t1_ref[...] = jnp.dot(zv, av_c * wa, preferred_element_type=jnp.float32)
        wz1_ref[...] = ae_c * wb
        c1_ref[...] = (jnp.sum(gv_c * wa, axis=0, keepdims=True)
                       + jnp.sum(ge_c * wb, axis=0, keepdims=True)
                       + vub_ref[...])
        wa2 = edw_ref[:D, :]
        wb2 = edw_ref[D:, :]
        t2_ref[...] = jnp.dot(zv, av_c * wb2, preferred_element_type=jnp.float32)
        wz2_ref[...] = ae_c * wa2
        c2_ref[...] = (jnp.sum(gv_c * wb2, axis=0, keepdims=True)
                       + jnp.sum(ge_c * wa2, axis=0, keepdims=True)
                       + edb_ref[...])
        aege_ref[...] = jnp.concatenate([ae, ge], axis=0)
    return body


def _tc_prep(zv, avgv, ste, pg, pbe, vuw, vub, edw, edb, first):
    outs = [
        jax.ShapeDtypeStruct((N, D), jnp.float32),
        jax.ShapeDtypeStruct((N, D), jnp.float32),
        jax.ShapeDtypeStruct((D, D), jnp.float32),
        jax.ShapeDtypeStruct((D, D), jnp.float32),
        jax.ShapeDtypeStruct((1, D), jnp.float32),
        jax.ShapeDtypeStruct((1, D), jnp.float32),
        jax.ShapeDtypeStruct((2, D), jnp.float32),
    ]
    ins = [zv, avgv, ste, pg, pbe, vuw, vub, edw, edb]
    return pl.pallas_call(
        _tc_prep_body(first),
        grid=(),
        in_specs=[_full(a.shape) for a in ins],
        out_specs=[_full(o.shape) for o in outs],
        out_shape=outs,
    )(*ins)


def _tc_big_body(ze_ref, wz1_ref, wz2_ref, c1_ref, c2_ref, z1_ref, z2_ref):
    ze = ze_ref[...]
    z1_ref[...] = jnp.dot(ze, wz1_ref[...],
                          preferred_element_type=jnp.float32) + c1_ref[...]
    z2_ref[...] = jnp.dot(ze, wz2_ref[...],
                          preferred_element_type=jnp.float32) + c2_ref[...]


def _tc_big(ze, wz1, wz2, c1, c2):
    outs = [jax.ShapeDtypeStruct((E, D), jnp.float32)] * 2
    return pl.pallas_call(
        _tc_big_body,
        grid=(NEBLK,),
        in_specs=[
            pl.BlockSpec((EBLK, D), lambda i: (i, 0)),
            pl.BlockSpec((D, D), lambda i: (0, 0)),
            pl.BlockSpec((D, D), lambda i: (0, 0)),
            pl.BlockSpec((1, D), lambda i: (0, 0)),
            pl.BlockSpec((1, D), lambda i: (0, 0)),
        ],
        out_specs=[pl.BlockSpec((EBLK, D), lambda i: (i, 0))] * 2,
        out_shape=outs,
    )(ze, wz1, wz2, c1, c2)


def _tc_mid_body(zv_ref, avgv_ref, sv2_ref, cntv_ref, st1_ref, st2_ref,
                 aege_ref, vug_ref, vube_ref, edg_ref, edbe_ref,
                 w1_ref, b1_ref, w2_ref, b2_ref, vupg_ref, vupbe_ref,
                 zvn_ref, avgvn_ref, epack_ref):
    s_v = sv2_ref[0] + sv2_ref[1]
    a1, k1 = _bn_affine(st1_ref[...], vug_ref[...], vube_ref[...], E)
    cnt = cntv_ref[:, 0:1] + cntv_ref[:, 1:2]
    agg_v = a1 * s_v + cnt * k1
    av_r = _c2r(avgv_ref[:, 0:1])
    gv_r = _c2r(avgv_ref[:, 1:2])
    hv = av_r * zv_ref[...] + gv_r + agg_v
    t = jnp.maximum(jnp.dot(hv, w1_ref[...],
                            preferred_element_type=jnp.float32)
                    + b1_ref[...], 0.0)
    zn = jnp.maximum(jnp.dot(t, w2_ref[...],
                             preferred_element_type=jnp.float32)
                     + b2_ref[...], 0.0)
    zvn_ref[...] = zn
    mv = jnp.sum(zn, axis=0, keepdims=True) / N
    vv = jnp.sum(zn * zn, axis=0, keepdims=True) / N - mv * mv
    avn = vupg_ref[...] / jnp.sqrt(vv + EPS)
    gvn = vupbe_ref[...] - avn * mv
    avgvn_ref[...] = jnp.concatenate([_r2c(avn), _r2c(gvn)], axis=1)
    a2, k2 = _bn_affine(st2_ref[...], edg_ref[...], edbe_ref[...], E)
    epack_ref[...] = jnp.concatenate([aege_ref[...], a2, k2], axis=0)


def _tc_mid(zv, avgv, sv2, cntv, st1, st2, aege, vug, vube, edg, edbe,
            w1, b1, w2, b2, vupg, vupbe):
    outs = [
        jax.ShapeDtypeStruct((N, D), jnp.float32),
        jax.ShapeDtypeStruct((D, 2), jnp.float32),
        jax.ShapeDtypeStruct((4, D), jnp.float32),
    ]
    ins = [zv, avgv, sv2, cntv, st1, st2, aege, vug, vube, edg, edbe,
           w1, b1, w2, b2, vupg, vupbe]
    return pl.pallas_call(
        _tc_mid_body,
        grid=(),
        in_specs=[_full(a.shape) for a in ins],
        out_specs=[_full(o.shape) for o in outs],
        out_shape=outs,
    )(*ins)


def _tc_emlp_body(final):
    def body(*refs):
        if final:
            (ze_ref, se_ref, cnte_ref, epack_ref, w1_ref, b1_ref, w2_ref,
             b2_ref, eb3_ref, zen_ref, st_ref, p1_ref) = refs
        else:
            (ze_ref, se_ref, cnte_ref, epack_ref, w1_ref, b1_ref, w2_ref,
             b2_ref, zen_ref, st_ref) = refs
        ae = epack_ref[0:1, :]
        ge = epack_ref[1:2, :]
        a2 = epack_ref[2:3, :]
        k2 = epack_ref[3:4, :]
        cnt = cnte_ref[:, 0:1] + cnte_ref[:, 1:2]
        he = ae * ze_ref[...] + ge + a2 * se_ref[...] + cnt * k2
        t = jnp.maximum(jnp.dot(he, w1_ref[...],
                                preferred_element_type=jnp.float32)
                        + b1_ref[...], 0.0)
        zn = jnp.maximum(jnp.dot(t, w2_ref[...],
                                 preferred_element_type=jnp.float32)
                         + b2_ref[...], 0.0)
        zen_ref[...] = zn
        cs = jnp.sum(zn, axis=0, keepdims=True)
        cq = jnp.sum(zn * zn, axis=0, keepdims=True)
        st_ref[0] = jnp.concatenate([cs, cq], axis=1)
        if final:
            ebrow = eb3_ref[0]
            bi = lax.broadcasted_iota(jnp.int32, (B, EBLK), 0)
            oh = jnp.where(bi == ebrow, 1.0, 0.0)
            p1_ref[0] = jnp.dot(oh, zn, preferred_element_type=jnp.float32)
    return body


def _tc_emlp(ze, se, cnte, epack, w1, b1, w2, b2, eb3=None):
    final = eb3 is not None
    outs = [
        jax.ShapeDtypeStruct((E, D), jnp.float32),
        jax.ShapeDtypeStruct((NEBLK, 1, 2 * D), jnp.float32),
    ]
    out_specs = [
        pl.BlockSpec((EBLK, D), lambda i: (i, 0)),
        pl.BlockSpec((1, 1, 2 * D), lambda i: (i, 0, 0)),
    ]
    in_specs = [
        pl.BlockSpec((EBLK, D), lambda i: (i, 0)),
        pl.BlockSpec((EBLK, D), lambda i: (i, 0)),
        pl.BlockSpec((EBLK, 2), lambda i: (i, 0)),
        pl.BlockSpec((4, D), lambda i: (0, 0)),
        pl.BlockSpec((D, D), lambda i: (0, 0)),
        pl.BlockSpec((1, D), lambda i: (0, 0)),
        pl.BlockSpec((D, D), lambda i: (0, 0)),
        pl.BlockSpec((1, D), lambda i: (0, 0)),
    ]
    ins = [ze, se, cnte, epack, w1, b1, w2, b2]
    if final:
        outs.append(jax.ShapeDtypeStruct((NEBLK, B, D), jnp.float32))
        out_specs.append(pl.BlockSpec((1, B, D), lambda i: (i, 0, 0)))
        in_specs.append(pl.BlockSpec((1, 1, EBLK), lambda i: (i, 0, 0)))
        ins.append(eb3)
    return pl.pallas_call(
        _tc_emlp_body(final),
        grid=(NEBLK,),
        in_specs=in_specs,
        out_specs=out_specs,
        out_shape=outs,
    )(*ins)


def _tc_head_body(zv_ref, avgv_ref, ste_ref, p1p_ref, b2d_ref, eb2d_ref,
                  eg_ref, ebe_ref, l1w_ref, l1b_ref, l2w_ref, l2b_ref,
                  out_ref):
    ae, ge = _bn_affine(ste_ref[...], eg_ref[...], ebe_ref[...], E)
    av = _c2r(avgv_ref[:, 0:1])
    gv = _c2r(avgv_ref[:, 1:2])
    brow = b2d_ref[0:1, :]
    bi = lax.broadcasted_iota(jnp.int32, (B, N), 0)
    oh = jnp.where(bi == brow, 1.0, 0.0)
    p0z = jnp.dot(oh, zv_ref[...], preferred_element_type=jnp.float32)
    cntb0 = jnp.sum(oh, axis=1, keepdims=True)

    def cb(k, acc):
        sl = eb2d_ref[0:1, pl.ds(k * 16000, 16000)]
        bi2 = lax.broadcasted_iota(jnp.int32, (B, 16000), 0)
        return acc + jnp.sum(jnp.where(bi2 == sl, 1.0, 0.0),
                             axis=1, keepdims=True)

    cntb1 = lax.fori_loop(0, E // 16000, cb, jnp.zeros((B, 1), jnp.float32))
    p0 = av * p0z + cntb0 * gv
    p1 = ae * jnp.sum(p1p_ref[...], axis=0) + cntb1 * ge
    h = jnp.maximum(jnp.dot(p0 + p1, l1w_ref[...],
                            preferred_element_type=jnp.float32)
                    + l1b_ref[...], 0.0)
    out_ref[...] = jnp.dot(h, l2w_ref[...],
                           preferred_element_type=jnp.float32) + l2b_ref[...]


def _tc_head(zv, avgv, ste, p1p, b2d, eb2d, eg, ebe, l1w, l1b, l2w, l2b):
    ins = [zv, avgv, ste, p1p, b2d, eb2d, eg, ebe, l1w, l1b, l2w, l2b]
    out = jax.ShapeDtypeStruct((B, NCLS), jnp.float32)
    return pl.pallas_call(
        _tc_head_body,
        grid=(),
        in_specs=[_full(a.shape) for a in ins],
        out_specs=_full((B, NCLS)),
        out_shape=out,
    )(*ins)


def kernel(x, edge_attr, params, edge_index, edge_adj, batch, edge_batch):
    u = edge_index[0].astype(jnp.int32)
    v = edge_index[1].astype(jnp.int32)
    es = edge_adj[0].astype(jnp.int32)
    ed = edge_adj[1].astype(jnp.int32)

    sv, cntv2, cnte2 = _sc_prep_k()(u, ed, v)
    cntv = cntv2.reshape(NCORES, N).T
    cnte = cnte2.reshape(NCORES, E).T

    def row(w):
        return w.reshape(1, w.shape[0])

    zv = x
    ze = edge_attr
    avgv = jnp.concatenate([jnp.ones((D, 1), jnp.float32),
                            jnp.zeros((D, 1), jnp.float32)], axis=1)
    ste = jnp.zeros((NEBLK, 1, 2 * D), jnp.float32)
    eb3 = edge_batch.astype(jnp.int32).reshape(NEBLK, 1, EBLK)
    b2d = batch.astype(jnp.int32).reshape(1, N)
    eb2d = edge_batch.astype(jnp.int32).reshape(1, E)

    for l in range(LAYERS):
        p = params['l%d' % l]
        if l == 0:
            pg = jnp.ones((1, D), jnp.float32)
            pbe = jnp.zeros((1, D), jnp.float32)
        else:
            pp = params['l%d' % (l - 1)]
            pg = row(pp['eup_g'])
            pbe = row(pp['eup_be'])
        t1, t2, wz1, wz2, c1, c2, aege = _tc_prep(
            zv, avgv, ste, pg, pbe, p['vu_W'], row(p['vu_b']),
            p['ed_W'], row(p['ed_b']), first=(l == 0))
        z1, z2 = _tc_big(ze, wz1, wz2, c1, c2)
        sv2, st1 = _sc_y1_k()(z1, t1, u, v)
        y2, st2 = _sc_y2_k()(z2, t2, es, sv)
        se = _sc_scatter_e_k()(y2, ed)
        zv, avgv, epack = _tc_mid(
            zv, avgv, sv2, cntv, st1, st2, aege,
            row(p['vu_g']), row(p['vu_be']), row(p['ed_g']), row(p['ed_be']),
            p['vup_W1'], row(p['vup_b1']), p['vup_W2'], row(p['vup_b2']),
            row(p['vup_g']), row(p['vup_be']))
        if l < LAYERS - 1:
            ze, ste = _tc_emlp(ze, se, cnte, epack,
                               p['eup_W1'], row(p['eup_b1']),
                               p['eup_W2'], row(p['eup_b2']))
        else:
            ze, ste, p1p = _tc_emlp(ze, se, cnte, epack,
                                    p['eup_W1'], row(p['eup_b1']),
                                    p['eup_W2'], row(p['eup_b2']), eb3)

    pl1 = params['l%d' % (LAYERS - 1)]
    return _tc_head(zv, avgv, ste, p1p, b2d, eb2d,
                    row(pl1['eup_g']), row(pl1['eup_be']),
                    params['lin1_W'], row(params['lin1_b']),
                    params['lin2_W'], row(params['lin2_b']))
